# Initial kernel scaffold; baseline (speedup 1.0000x reference)
import functools

import jax
import jax.numpy as jnp
from jax import lax
from jax.experimental import pallas as pl
from jax.experimental.pallas import tpu as pltpu
from jax.experimental.pallas import tpu_sc as plsc

N = 10000
NP = 10240
E = 320000
D = 128
NC = 2
NS = 16
NW = NC * NS
CH = 80

_mesh = functools.partial(
    plsc.VectorSubcoreMesh, core_axis_name="c", subcore_axis_name="s",
    num_cores=NC, num_subcores=NS)


def _ids():
    cid = lax.axis_index("c")
    sid = lax.axis_index("s")
    return cid, sid


def _fill_stripe(stripe_ref, acc_ref, start, size, value):
    vv = jnp.full((16,), value, jnp.float32)

    def body(i, _):
        stripe_ref[pl.ds(i * 16, 16)] = vv
        return 0

    lax.fori_loop(0, size // 16, body, 0)
    pltpu.sync_copy(stripe_ref, acc_ref.at[pl.ds(start, size)])


def _k1_body(col_hbm, ew_hbm, out_hbm, col_t, ew_t, cbuf, stripe, acc):
    cid, sid = _ids()
    ept = E // NW
    base = (cid * NS + sid) * ept
    pltpu.sync_copy(col_hbm.at[pl.ds(base, ept)], col_t)
    pltpu.sync_copy(ew_hbm.at[pl.ds(base, ept)], ew_t)

    stripe_n = NP // NS
    init = jnp.where(cid == 0, 1.0, 0.0)
    _fill_stripe(stripe, acc, sid * stripe_n, stripe_n, init)
    plsc.subcore_barrier()

    def chunk(i, _):
        off = i * CH
        for g in range(CH // 16):
            cbuf[pl.ds(g * 16, 16)] = col_t[pl.ds(off + g * 16, 16)]
        pltpu.sync_copy(ew_t.at[pl.ds(off, CH)], acc.at[cbuf], add=True)
        return 0

    lax.fori_loop(0, ept // CH, chunk, 0)
    plsc.subcore_barrier()
    pltpu.sync_copy(acc.at[pl.ds(sid * stripe_n, stripe_n)],
                    out_hbm.at[cid, pl.ds(sid * stripe_n, stripe_n)])


def _k1(col, ew):
    return pl.kernel(
        _k1_body,
        out_type=jax.ShapeDtypeStruct((NC, NP), jnp.float32),
        mesh=_mesh(),
        scratch_types=[
            pltpu.VMEM((E // NW,), jnp.int32),
            pltpu.VMEM((E // NW,), jnp.float32),
            pltpu.VMEM((CH,), jnp.int32),
            pltpu.VMEM((NP // NS,), jnp.float32),
            pltpu.VMEM_SHARED((NP,), jnp.float32),
        ],
    )(col, ew)


def _k2_body(x_ref, w_ref, d_ref, g1_ref, dv_ref):
    deg = d_ref[0] + d_ref[1]
    dv = jnp.where(deg > 0.0,
                   lax.rsqrt(jnp.maximum(deg, 1e-30)), 0.0)
    dv_ref[...] = dv
    h = jnp.dot(x_ref[...], w_ref[...], preferred_element_type=jnp.float32)
    g1_ref[...] = h * dv


def _k2(x_p, W1, degp):
    return pl.pallas_call(
        _k2_body,
        out_shape=[
            jax.ShapeDtypeStruct((NP, D), jnp.float32),
            jax.ShapeDtypeStruct((NP, 1), jnp.float32),
        ],
    )(x_p, W1, degp.reshape(NC, NP, 1))


def _k3_body(g1_hbm, row_hbm, col_hbm, ew_hbm, out_hbm,
             row_t, col_t, ew_t, cbuf, gbuf, acc):
    cid, sid = _ids()
    ept = E // NW
    base = (cid * NS + sid) * ept
    pltpu.sync_copy(row_hbm.at[pl.ds(base, ept)], row_t)
    pltpu.sync_copy(col_hbm.at[pl.ds(base, ept)], col_t)
    pltpu.sync_copy(ew_hbm.at[pl.ds(base, ept)], ew_t)

    zv = jnp.zeros((16,), jnp.float32)

    def zb(e, _):
        for j in range(D // 16):
            gbuf[e, pl.ds(j * 16, 16)] = zv
        return 0

    lax.fori_loop(0, CH, zb, 0)
    stripe_n = NP // NS
    for k in range(stripe_n // CH):
        pltpu.sync_copy(gbuf, acc.at[pl.ds(sid * stripe_n + k * CH, CH)])
    plsc.subcore_barrier()

    def chunk(i, _):
        off = i * CH
        for g in range(CH // 16):
            cbuf[pl.ds(g * 16, 16)] = col_t[pl.ds(off + g * 16, 16)]
        pltpu.sync_copy(g1_hbm.at[row_t.at[pl.ds(off, CH)]], gbuf)

        def se(e, _):
            cv = jnp.full((16,), ew_t[off + e], jnp.float32)
            for j in range(D // 16):
                gbuf[e, pl.ds(j * 16, 16)] = gbuf[e, pl.ds(j * 16, 16)] * cv
            return 0

        lax.fori_loop(0, CH, se, 0)
        pltpu.sync_copy(gbuf, acc.at[cbuf], add=True)
        return 0

    lax.fori_loop(0, ept // CH, chunk, 0)
    plsc.subcore_barrier()
    pltpu.sync_copy(acc.at[pl.ds(sid * stripe_n, stripe_n)],
                    out_hbm.at[cid, pl.ds(sid * stripe_n, stripe_n)])


def _k3(g1, row, col, ew):
    return pl.kernel(
        _k3_body,
        out_type=jax.ShapeDtypeStruct((NC, NP, D), jnp.float32),
        mesh=_mesh(),
        scratch_types=[
            pltpu.VMEM((E // NW,), jnp.int32),
            pltpu.VMEM((E // NW,), jnp.int32),
            pltpu.VMEM((E // NW,), jnp.float32),
            pltpu.VMEM((CH,), jnp.int32),
            pltpu.VMEM((CH, D), jnp.float32),
            pltpu.VMEM_SHARED((NP, D), jnp.float32),
        ],
    )(g1, row, col, ew)


def _k4_body(t_ref, g1_ref, dv_ref, w2_ref, b1_ref, m2_ref):
    t = t_ref[0] + t_ref[1] + g1_ref[...]
    a1 = jnp.maximum(dv_ref[...] * t + b1_ref[...], 0.0)
    h2 = jnp.sum(a1 * w2_ref[...], axis=1, keepdims=True)
    m2_ref[...] = dv_ref[...] * h2


def _k4(t1p, g1, dv, W2, b1):
    return pl.pallas_call(
        _k4_body,
        out_shape=jax.ShapeDtypeStruct((NP, 1), jnp.float32),
    )(t1p, g1, dv, W2.reshape(1, D), b1.reshape(1, D))


def _k5_body(m2_hbm, dv_hbm, row_hbm, col_hbm, ew_hbm, b2_hbm, out_hbm,
             m2_t, row_t, col_t, ew_t, cbuf, ubuf, tbuf, dbuf, b2t, acc):
    cid, sid = _ids()

    @pl.when(cid == 0)
    def _():
        ept = E // NS
        base = sid * ept
        pltpu.sync_copy(m2_hbm, m2_t)
        pltpu.sync_copy(row_hbm.at[pl.ds(base, ept)], row_t)
        pltpu.sync_copy(col_hbm.at[pl.ds(base, ept)], col_t)
        pltpu.sync_copy(ew_hbm.at[pl.ds(base, ept)], ew_t)

        stripe_n = NP // NS
        _fill_stripe(tbuf, acc, sid * stripe_n, stripe_n, 0.0)
        plsc.subcore_barrier()

        def chunk(i, _):
            off = i * CH
            for g in range(CH // 16):
                rv = row_t[pl.ds(off + g * 16, 16)]
                mg = plsc.load_gather(m2_t, [rv])
                ubuf[pl.ds(g * 16, 16)] = mg * ew_t[pl.ds(off + g * 16, 16)]
                cbuf[pl.ds(g * 16, 16)] = col_t[pl.ds(off + g * 16, 16)]
            pltpu.sync_copy(ubuf, acc.at[cbuf], add=True)
            return 0

        lax.fori_loop(0, ept // CH, chunk, 0)
        plsc.subcore_barrier()

        sb = sid * stripe_n
        pltpu.sync_copy(acc.at[pl.ds(sb, stripe_n)], tbuf)
        pltpu.sync_copy(dv_hbm.at[pl.ds(sb, stripe_n)], dbuf)
        pltpu.sync_copy(b2_hbm, b2t)
        b2v = b2t[...]

        def fin(g, _):
            o = dbuf[pl.ds(g * 16, 16)] * (
                tbuf[pl.ds(g * 16, 16)] + m2_t[pl.ds(sb + g * 16, 16)]) + b2v
            tbuf[pl.ds(g * 16, 16)] = o
            return 0

        lax.fori_loop(0, stripe_n // 16, fin, 0)
        pltpu.sync_copy(tbuf, out_hbm.at[pl.ds(sb, stripe_n)])


def _k5(m2, dv, row, col, ew, b2):
    return pl.kernel(
        _k5_body,
        out_type=jax.ShapeDtypeStruct((NP,), jnp.float32),
        mesh=_mesh(),
        scratch_types=[
            pltpu.VMEM((NP,), jnp.float32),
            pltpu.VMEM((E // NS,), jnp.int32),
            pltpu.VMEM((E // NS,), jnp.int32),
            pltpu.VMEM((E // NS,), jnp.float32),
            pltpu.VMEM((CH,), jnp.int32),
            pltpu.VMEM((CH,), jnp.float32),
            pltpu.VMEM((NP // NS,), jnp.float32),
            pltpu.VMEM((NP // NS,), jnp.float32),
            pltpu.VMEM((16,), jnp.float32),
            pltpu.VMEM_SHARED((NP,), jnp.float32),
        ],
    )(m2, dv, row, col, ew, jnp.broadcast_to(b2, (16,)))


def kernel(x, edge_index, edge_weight, W1, b1, W2, b2):
    row = edge_index[0]
    col = edge_index[1]
    x_p = jnp.pad(x, ((0, NP - N), (0, 0)))

    degp = _k1(col, edge_weight)
    g1, dv = _k2(x_p, W1, degp)
    t1p = _k3(g1, row, col, edge_weight)
    m2 = _k4(t1p, g1, dv, W2, b1)
    outp = _k5(m2.reshape(NP), dv.reshape(NP), row, col, edge_weight, b2)
    return outp[:N].reshape(N, 1)


# speedup vs baseline: 17.1451x; 17.1451x over previous
import functools

import jax
import jax.numpy as jnp
from jax import lax
from jax.experimental import pallas as pl
from jax.experimental.pallas import tpu as pltpu
from jax.experimental.pallas import tpu_sc as plsc

N = 10000
NP = 10240
E = 320000
D = 128
NC = 2
NS = 16
NW = NC * NS
CH = 80

_mesh = functools.partial(
    plsc.VectorSubcoreMesh, core_axis_name="c", subcore_axis_name="s",
    num_cores=NC, num_subcores=NS)


def _ids():
    cid = lax.axis_index("c")
    sid = lax.axis_index("s")
    return cid, sid


def _fill_stripe(stripe_ref, acc_ref, start, size, value):
    vv = jnp.full((16,), value, jnp.float32)

    def body(i, _):
        stripe_ref[pl.ds(i * 16, 16)] = vv
        return 0

    lax.fori_loop(0, size // 16, body, 0)
    pltpu.sync_copy(stripe_ref, acc_ref.at[pl.ds(start, size)])


def _k1_body(col_hbm, ew_hbm, out_hbm, col_t, ew_t, cbuf, stripe, acc):
    cid, sid = _ids()
    ept = E // NW
    base = (cid * NS + sid) * ept
    pltpu.sync_copy(col_hbm.at[pl.ds(base, ept)], col_t)
    pltpu.sync_copy(ew_hbm.at[pl.ds(base, ept)], ew_t)

    stripe_n = NP // NS
    init = jnp.where(cid == 0, 1.0, 0.0)
    _fill_stripe(stripe, acc, sid * stripe_n, stripe_n, init)
    plsc.subcore_barrier()

    def chunk(i, _):
        off = i * CH
        for g in range(CH // 16):
            cbuf[pl.ds(g * 16, 16)] = col_t[pl.ds(off + g * 16, 16)]
        pltpu.sync_copy(ew_t.at[pl.ds(off, CH)], acc.at[cbuf], add=True)
        return 0

    lax.fori_loop(0, ept // CH, chunk, 0)
    plsc.subcore_barrier()
    pltpu.sync_copy(acc.at[pl.ds(sid * stripe_n, stripe_n)],
                    out_hbm.at[cid, pl.ds(sid * stripe_n, stripe_n)])


def _k1(col, ew):
    return pl.kernel(
        _k1_body,
        out_type=jax.ShapeDtypeStruct((NC, NP), jnp.float32),
        mesh=_mesh(),
        scratch_types=[
            pltpu.VMEM((E // NW,), jnp.int32),
            pltpu.VMEM((E // NW,), jnp.float32),
            pltpu.VMEM((CH,), jnp.int32),
            pltpu.VMEM((NP // NS,), jnp.float32),
            pltpu.VMEM_SHARED((NP,), jnp.float32),
        ],
    )(col, ew)


def _k2_body(x_ref, w_ref, d_ref, g1_ref, dv_ref):
    deg = d_ref[0] + d_ref[1]
    dv = jnp.where(deg > 0.0,
                   lax.rsqrt(jnp.maximum(deg, 1e-30)), 0.0)
    dv_ref[...] = dv
    h = jnp.dot(x_ref[...], w_ref[...], preferred_element_type=jnp.float32)
    g1_ref[...] = h * dv


def _k2(x_p, W1, degp):
    return pl.pallas_call(
        _k2_body,
        out_shape=[
            jax.ShapeDtypeStruct((NP, D), jnp.float32),
            jax.ShapeDtypeStruct((NP, 1), jnp.float32),
        ],
    )(x_p, W1, degp.reshape(NC, NP, 1))


def _k3_body(g1_hbm, row_hbm, col_hbm, ew_hbm, out_hbm,
             row_t, col_t, ew_t, cbuf, gbuf, acc):
    cid, sid = _ids()
    ept = E // NW
    base = (cid * NS + sid) * ept
    pltpu.sync_copy(row_hbm.at[pl.ds(base, ept)], row_t)
    pltpu.sync_copy(col_hbm.at[pl.ds(base, ept)], col_t)
    pltpu.sync_copy(ew_hbm.at[pl.ds(base, ept)], ew_t.at[pl.ds(0, ept)])

    zv = jnp.zeros((16,), jnp.float32)

    def zb(e, _):
        for j in range(D // 16):
            gbuf[e, pl.ds(j * 16, 16)] = zv
        return 0

    lax.fori_loop(0, CH, zb, 0)
    stripe_n = NP // NS
    for k in range(stripe_n // CH):
        pltpu.sync_copy(gbuf, acc.at[pl.ds(sid * stripe_n + k * CH, CH)])
    plsc.subcore_barrier()

    def chunk(i, _):
        off = i * CH
        for g in range(CH // 16):
            cbuf[pl.ds(g * 16, 16)] = col_t[pl.ds(off + g * 16, 16)]
        pltpu.sync_copy(g1_hbm.at[row_t.at[pl.ds(off, CH)]], gbuf)

        def se(e, _):
            ev = ew_t[pl.ds(off + e, 16)]
            cv = jnp.full((16,), ev[0], jnp.float32)
            for j in range(D // 16):
                gbuf[e, pl.ds(j * 16, 16)] = gbuf[e, pl.ds(j * 16, 16)] * cv
            return 0

        lax.fori_loop(0, CH, se, 0)
        pltpu.sync_copy(gbuf, acc.at[cbuf], add=True)
        return 0

    lax.fori_loop(0, ept // CH, chunk, 0)
    plsc.subcore_barrier()
    pltpu.sync_copy(acc.at[pl.ds(sid * stripe_n, stripe_n)],
                    out_hbm.at[cid, pl.ds(sid * stripe_n, stripe_n)])


def _k3(g1, row, col, ew):
    return pl.kernel(
        _k3_body,
        out_type=jax.ShapeDtypeStruct((NC, NP, D), jnp.float32),
        mesh=_mesh(),
        scratch_types=[
            pltpu.VMEM((E // NW,), jnp.int32),
            pltpu.VMEM((E // NW,), jnp.int32),
            pltpu.VMEM((E // NW + 16,), jnp.float32),
            pltpu.VMEM((CH,), jnp.int32),
            pltpu.VMEM((CH, D), jnp.float32),
            pltpu.VMEM_SHARED((NP, D), jnp.float32),
        ],
    )(g1, row, col, ew)


def _k4_body(t_ref, g1_ref, dv_ref, w2_ref, b1_ref, m2_ref):
    t = t_ref[0] + t_ref[1] + g1_ref[...]
    a1 = jnp.maximum(dv_ref[...] * t + b1_ref[...], 0.0)
    h2 = jnp.sum(a1 * w2_ref[...], axis=1, keepdims=True)
    m2_ref[...] = dv_ref[...] * h2


def _k4(t1p, g1, dv, W2, b1):
    return pl.pallas_call(
        _k4_body,
        out_shape=jax.ShapeDtypeStruct((NP, 1), jnp.float32),
    )(t1p, g1, dv, W2.reshape(1, D), b1.reshape(1, D))


def _k5_body(m2_hbm, dv_hbm, row_hbm, col_hbm, ew_hbm, b2_hbm, out_hbm,
             mbuf, row_t, col_t, ew_t, cbuf, ubuf, tbuf, dbuf, b2t, acc):
    cid, sid = _ids()

    @pl.when(cid == 0)
    def _():
        ept = E // NS
        base = sid * ept
        pltpu.sync_copy(row_hbm.at[pl.ds(base, ept)], row_t)
        pltpu.sync_copy(col_hbm.at[pl.ds(base, ept)], col_t)
        pltpu.sync_copy(ew_hbm.at[pl.ds(base, ept)], ew_t)

        stripe_n = NP // NS
        _fill_stripe(tbuf, acc, sid * stripe_n, stripe_n, 0.0)
        plsc.subcore_barrier()

        def chunk(i, _):
            off = i * CH
            pltpu.sync_copy(m2_hbm.at[row_t.at[pl.ds(off, CH)]], ubuf)
            for g in range(CH // 16):
                ubuf[pl.ds(g * 16, 16)] = (
                    ubuf[pl.ds(g * 16, 16)] * ew_t[pl.ds(off + g * 16, 16)])
                cbuf[pl.ds(g * 16, 16)] = col_t[pl.ds(off + g * 16, 16)]
            pltpu.sync_copy(ubuf, acc.at[cbuf], add=True)
            return 0

        lax.fori_loop(0, ept // CH, chunk, 0)
        plsc.subcore_barrier()

        sb = sid * stripe_n
        pltpu.sync_copy(acc.at[pl.ds(sb, stripe_n)], tbuf)
        pltpu.sync_copy(dv_hbm.at[pl.ds(sb, stripe_n)], dbuf)
        pltpu.sync_copy(m2_hbm.at[pl.ds(sb, stripe_n)], mbuf)
        pltpu.sync_copy(b2_hbm, b2t)
        b2v = b2t[...]

        def fin(g, _):
            o = dbuf[pl.ds(g * 16, 16)] * (
                tbuf[pl.ds(g * 16, 16)] + mbuf[pl.ds(g * 16, 16)]) + b2v
            tbuf[pl.ds(g * 16, 16)] = o
            return 0

        lax.fori_loop(0, stripe_n // 16, fin, 0)
        pltpu.sync_copy(tbuf, out_hbm.at[pl.ds(sb, stripe_n)])


def _k5(m2, dv, row, col, ew, b2):
    return pl.kernel(
        _k5_body,
        out_type=jax.ShapeDtypeStruct((NP,), jnp.float32),
        mesh=_mesh(),
        scratch_types=[
            pltpu.VMEM((NP // NS,), jnp.float32),
            pltpu.VMEM((E // NS,), jnp.int32),
            pltpu.VMEM((E // NS,), jnp.int32),
            pltpu.VMEM((E // NS,), jnp.float32),
            pltpu.VMEM((CH,), jnp.int32),
            pltpu.VMEM((CH,), jnp.float32),
            pltpu.VMEM((NP // NS,), jnp.float32),
            pltpu.VMEM((NP // NS,), jnp.float32),
            pltpu.VMEM((16,), jnp.float32),
            pltpu.VMEM_SHARED((NP,), jnp.float32),
        ],
    )(m2, dv, row, col, ew, jnp.broadcast_to(b2, (16,)))


def kernel(x, edge_index, edge_weight, W1, b1, W2, b2):
    row = edge_index[0]
    col = edge_index[1]
    x_p = jnp.pad(x, ((0, NP - N), (0, 0)))

    degp = _k1(col, edge_weight)
    g1, dv = _k2(x_p, W1, degp)
    t1p = _k3(g1, row, col, edge_weight)
    m2 = _k4(t1p, g1, dv, W2, b1)
    outp = _k5(m2.reshape(NP), dv.reshape(NP), row, col, edge_weight, b2)
    return outp[:N].reshape(N, 1)


# speedup vs baseline: 23.6443x; 1.3791x over previous
import functools

import jax
import jax.numpy as jnp
from jax import lax
from jax.experimental import pallas as pl
from jax.experimental.pallas import tpu as pltpu
from jax.experimental.pallas import tpu_sc as plsc

N = 10000
NP = 10240
E = 320000
D = 128
NC = 2
NS = 16
NW = NC * NS
CH = 80
NBUF = 3

_mesh = functools.partial(
    plsc.VectorSubcoreMesh, core_axis_name="c", subcore_axis_name="s",
    num_cores=NC, num_subcores=NS)


def _ids():
    cid = lax.axis_index("c")
    sid = lax.axis_index("s")
    return cid, sid


def _fill_stripe(stripe_ref, acc_ref, start, size, value):
    vv = jnp.full((16,), value, jnp.float32)

    def body(i, _):
        stripe_ref[pl.ds(i * 16, 16)] = vv
        return 0

    lax.fori_loop(0, size // 16, body, 0)
    pltpu.sync_copy(stripe_ref, acc_ref.at[pl.ds(start, size)])


def _k1_body(col_hbm, ew_hbm, out_hbm, col_t, ew_t, cbuf, stripe, acc):
    cid, sid = _ids()
    ept = E // NW
    base = (cid * NS + sid) * ept
    pltpu.sync_copy(col_hbm.at[pl.ds(base, ept)], col_t)
    pltpu.sync_copy(ew_hbm.at[pl.ds(base, ept)], ew_t)

    stripe_n = NP // NS
    init = jnp.where(cid == 0, 1.0, 0.0)
    _fill_stripe(stripe, acc, sid * stripe_n, stripe_n, init)
    plsc.subcore_barrier()

    def chunk(i, _):
        off = i * CH
        for g in range(CH // 16):
            cbuf[pl.ds(g * 16, 16)] = col_t[pl.ds(off + g * 16, 16)]
        pltpu.sync_copy(ew_t.at[pl.ds(off, CH)], acc.at[cbuf], add=True)
        return 0

    lax.fori_loop(0, ept // CH, chunk, 0)
    plsc.subcore_barrier()
    pltpu.sync_copy(acc.at[pl.ds(sid * stripe_n, stripe_n)],
                    out_hbm.at[cid, pl.ds(sid * stripe_n, stripe_n)])


def _k1(col, ew):
    return pl.kernel(
        _k1_body,
        out_type=jax.ShapeDtypeStruct((NC, NP), jnp.float32),
        mesh=_mesh(),
        scratch_types=[
            pltpu.VMEM((E // NW,), jnp.int32),
            pltpu.VMEM((E // NW,), jnp.float32),
            pltpu.VMEM((CH,), jnp.int32),
            pltpu.VMEM((NP // NS,), jnp.float32),
            pltpu.VMEM_SHARED((NP,), jnp.float32),
        ],
    )(col, ew)


def _k2_body(x_ref, w_ref, d_ref, g1_ref, dv_ref):
    deg = d_ref[0] + d_ref[1]
    dv = jnp.where(deg > 0.0,
                   lax.rsqrt(jnp.maximum(deg, 1e-30)), 0.0)
    dv_ref[...] = dv
    h = jnp.dot(x_ref[...], w_ref[...], preferred_element_type=jnp.float32)
    g1_ref[...] = h * dv


def _k2(x_p, W1, degp):
    return pl.pallas_call(
        _k2_body,
        out_shape=[
            jax.ShapeDtypeStruct((NP, D), jnp.float32),
            jax.ShapeDtypeStruct((NP, 1), jnp.float32),
        ],
    )(x_p, W1, degp.reshape(NC, NP, 1))


def _k3_body(g1_hbm, row_hbm, col_hbm, ew_hbm, out_hbm,
             row_t, cbufs, ebufs, gbufs, acc, gsems, ssems, isems):
    cid, sid = _ids()
    ept = E // NW
    nch = ept // CH
    base = (cid * NS + sid) * ept
    pltpu.sync_copy(row_hbm.at[pl.ds(base, ept)], row_t)

    zv = jnp.zeros((16,), jnp.float32)

    def zb(e, _):
        for j in range(D // 16):
            gbufs[0][e, pl.ds(j * 16, 16)] = zv
        return 0

    lax.fori_loop(0, CH, zb, 0)
    stripe_n = NP // NS
    for k in range(stripe_n // CH):
        pltpu.sync_copy(gbufs[0], acc.at[pl.ds(sid * stripe_n + k * CH, CH)])
    plsc.subcore_barrier()

    def issue_icopy(k, b):
        off = base + k * CH
        pltpu.async_copy(col_hbm.at[pl.ds(off, CH)], cbufs[b], isems[b])
        pltpu.async_copy(ew_hbm.at[pl.ds(off, CH)],
                         ebufs[b].at[pl.ds(0, CH)], isems[b])

    def wait_icopy(b):
        pltpu.make_async_copy(col_hbm.at[pl.ds(0, CH)], cbufs[b],
                              isems[b]).wait()
        pltpu.make_async_copy(ew_hbm.at[pl.ds(0, CH)],
                              ebufs[b].at[pl.ds(0, CH)], isems[b]).wait()

    def issue_gather(k, b):
        pltpu.async_copy(g1_hbm.at[row_t.at[pl.ds(k * CH, CH)]],
                         gbufs[b], gsems[b])

    def wait_gather(b):
        pltpu.make_async_copy(g1_hbm.at[pl.ds(0, CH)], gbufs[b],
                              gsems[b]).wait()

    def issue_scatter(b):
        pltpu.async_copy(gbufs[b], acc.at[cbufs[b]], ssems[b], add=True)

    def wait_scatter(b):
        pltpu.make_async_copy(gbufs[b], acc.at[pl.ds(0, CH)], ssems[b]).wait()

    def compute(b):
        def se(eq, _):
            for u in range(4):
                e = eq * 4 + u
                ev = ebufs[b][pl.ds(e, 16)]
                cv = jnp.full((16,), ev[0], jnp.float32)
                for j in range(D // 16):
                    gbufs[b][e, pl.ds(j * 16, 16)] = (
                        gbufs[b][e, pl.ds(j * 16, 16)] * cv)
            return 0

        lax.fori_loop(0, CH // 4, se, 0)

    for b in range(NBUF - 1):
        issue_icopy(b, b)
        issue_gather(b, b)

    def slot(k, _):
        for b in range(NBUF):
            @pl.when(k % NBUF == b)
            def _():
                br = (b + NBUF - 1) % NBUF
                wait_gather(b)
                wait_icopy(b)
                compute(b)
                issue_scatter(b)

                @pl.when(k + NBUF - 1 <= nch - 1)
                def _():
                    @pl.when(k >= 1)
                    def _():
                        wait_scatter(br)
                    issue_icopy(k + NBUF - 1, br)
                    issue_gather(k + NBUF - 1, br)
        return 0

    lax.fori_loop(0, nch, slot, 0)
    for b in range(NBUF):
        wait_scatter(b)
    plsc.subcore_barrier()
    pltpu.sync_copy(acc.at[pl.ds(sid * stripe_n, stripe_n)],
                    out_hbm.at[cid, pl.ds(sid * stripe_n, stripe_n)])


def _k3(g1, row, col, ew):
    return pl.kernel(
        _k3_body,
        out_type=jax.ShapeDtypeStruct((NC, NP, D), jnp.float32),
        mesh=_mesh(),
        scratch_types=[
            pltpu.VMEM((E // NW,), jnp.int32),
            tuple(pltpu.VMEM((CH,), jnp.int32) for _ in range(NBUF)),
            tuple(pltpu.VMEM((CH + 16,), jnp.float32) for _ in range(NBUF)),
            tuple(pltpu.VMEM((CH, D), jnp.float32) for _ in range(NBUF)),
            pltpu.VMEM_SHARED((NP, D), jnp.float32),
            tuple(pltpu.SemaphoreType.DMA for _ in range(NBUF)),
            tuple(pltpu.SemaphoreType.DMA for _ in range(NBUF)),
            tuple(pltpu.SemaphoreType.DMA for _ in range(NBUF)),
        ],
    )(g1, row, col, ew)


def _k4_body(t_ref, g1_ref, dv_ref, w2_ref, b1_ref, m2_ref):
    t = t_ref[0] + t_ref[1] + g1_ref[...]
    a1 = jnp.maximum(dv_ref[...] * t + b1_ref[...], 0.0)
    h2 = jnp.sum(a1 * w2_ref[...], axis=1, keepdims=True)
    m2_ref[...] = dv_ref[...] * h2


def _k4(t1p, g1, dv, W2, b1):
    return pl.pallas_call(
        _k4_body,
        out_shape=jax.ShapeDtypeStruct((NP, 1), jnp.float32),
    )(t1p, g1, dv, W2.reshape(1, D), b1.reshape(1, D))


def _k5_body(m2_hbm, dv_hbm, row_hbm, col_hbm, ew_hbm, b2_hbm, out_hbm,
             mbuf, row_t, col_t, ew_t, cbuf, ubuf, tbuf, dbuf, b2t, acc):
    cid, sid = _ids()

    @pl.when(cid == 0)
    def _():
        ept = E // NS
        base = sid * ept
        pltpu.sync_copy(row_hbm.at[pl.ds(base, ept)], row_t)
        pltpu.sync_copy(col_hbm.at[pl.ds(base, ept)], col_t)
        pltpu.sync_copy(ew_hbm.at[pl.ds(base, ept)], ew_t)

        stripe_n = NP // NS
        _fill_stripe(tbuf, acc, sid * stripe_n, stripe_n, 0.0)
        plsc.subcore_barrier()

        def chunk(i, _):
            off = i * CH
            pltpu.sync_copy(m2_hbm.at[row_t.at[pl.ds(off, CH)]], ubuf)
            for g in range(CH // 16):
                ubuf[pl.ds(g * 16, 16)] = (
                    ubuf[pl.ds(g * 16, 16)] * ew_t[pl.ds(off + g * 16, 16)])
                cbuf[pl.ds(g * 16, 16)] = col_t[pl.ds(off + g * 16, 16)]
            pltpu.sync_copy(ubuf, acc.at[cbuf], add=True)
            return 0

        lax.fori_loop(0, ept // CH, chunk, 0)
        plsc.subcore_barrier()

        sb = sid * stripe_n
        pltpu.sync_copy(acc.at[pl.ds(sb, stripe_n)], tbuf)
        pltpu.sync_copy(dv_hbm.at[pl.ds(sb, stripe_n)], dbuf)
        pltpu.sync_copy(m2_hbm.at[pl.ds(sb, stripe_n)], mbuf)
        pltpu.sync_copy(b2_hbm, b2t)
        b2v = b2t[...]

        def fin(g, _):
            o = dbuf[pl.ds(g * 16, 16)] * (
                tbuf[pl.ds(g * 16, 16)] + mbuf[pl.ds(g * 16, 16)]) + b2v
            tbuf[pl.ds(g * 16, 16)] = o
            return 0

        lax.fori_loop(0, stripe_n // 16, fin, 0)
        pltpu.sync_copy(tbuf, out_hbm.at[pl.ds(sb, stripe_n)])


def _k5(m2, dv, row, col, ew, b2):
    return pl.kernel(
        _k5_body,
        out_type=jax.ShapeDtypeStruct((NP,), jnp.float32),
        mesh=_mesh(),
        scratch_types=[
            pltpu.VMEM((NP // NS,), jnp.float32),
            pltpu.VMEM((E // NS,), jnp.int32),
            pltpu.VMEM((E // NS,), jnp.int32),
            pltpu.VMEM((E // NS,), jnp.float32),
            pltpu.VMEM((CH,), jnp.int32),
            pltpu.VMEM((CH,), jnp.float32),
            pltpu.VMEM((NP // NS,), jnp.float32),
            pltpu.VMEM((NP // NS,), jnp.float32),
            pltpu.VMEM((16,), jnp.float32),
            pltpu.VMEM_SHARED((NP,), jnp.float32),
        ],
    )(m2, dv, row, col, ew, jnp.broadcast_to(b2, (16,)))


def kernel(x, edge_index, edge_weight, W1, b1, W2, b2):
    row = edge_index[0]
    col = edge_index[1]
    x_p = jnp.pad(x, ((0, NP - N), (0, 0)))

    degp = _k1(col, edge_weight)
    g1, dv = _k2(x_p, W1, degp)
    t1p = _k3(g1, row, col, edge_weight)
    m2 = _k4(t1p, g1, dv, W2, b1)
    outp = _k5(m2.reshape(NP), dv.reshape(NP), row, col, edge_weight, b2)
    return outp[:N].reshape(N, 1)


# speedup vs baseline: 35.2426x; 1.4905x over previous
import functools

import jax
import jax.numpy as jnp
from jax import lax
from jax.experimental import pallas as pl
from jax.experimental.pallas import tpu as pltpu
from jax.experimental.pallas import tpu_sc as plsc

N = 10000
NP = 10240
E = 320000
D = 128
NC = 2
NS = 16
NW = NC * NS
CH = 80
NBUF = 3

_mesh = functools.partial(
    plsc.VectorSubcoreMesh, core_axis_name="c", subcore_axis_name="s",
    num_cores=NC, num_subcores=NS)


def _ids():
    cid = lax.axis_index("c")
    sid = lax.axis_index("s")
    return cid, sid


def _fill_stripe(stripe_ref, acc_ref, start, size, value):
    vv = jnp.full((16,), value, jnp.float32)

    def body(i, _):
        stripe_ref[pl.ds(i * 16, 16)] = vv
        return 0

    lax.fori_loop(0, size // 16, body, 0)
    pltpu.sync_copy(stripe_ref, acc_ref.at[pl.ds(start, size)])


def _k1_body(col_hbm, ew_hbm, out_hbm, col_t, ew_t, cbuf, stripe, acc):
    cid, sid = _ids()
    ept = E // NW
    base = (cid * NS + sid) * ept
    pltpu.sync_copy(col_hbm.at[pl.ds(base, ept)], col_t)
    pltpu.sync_copy(ew_hbm.at[pl.ds(base, ept)], ew_t)

    stripe_n = NP // NS
    init = jnp.where(cid == 0, 1.0, 0.0)
    _fill_stripe(stripe, acc, sid * stripe_n, stripe_n, init)
    plsc.subcore_barrier()

    def chunk(i, _):
        off = i * CH
        for g in range(CH // 16):
            cbuf[pl.ds(g * 16, 16)] = col_t[pl.ds(off + g * 16, 16)]
        pltpu.sync_copy(ew_t.at[pl.ds(off, CH)], acc.at[cbuf], add=True)
        return 0

    lax.fori_loop(0, ept // CH, chunk, 0)
    plsc.subcore_barrier()
    pltpu.sync_copy(acc.at[pl.ds(sid * stripe_n, stripe_n)],
                    out_hbm.at[cid, pl.ds(sid * stripe_n, stripe_n)])


def _k1(col, ew):
    return pl.kernel(
        _k1_body,
        out_type=jax.ShapeDtypeStruct((NC, NP), jnp.float32),
        mesh=_mesh(),
        scratch_types=[
            pltpu.VMEM((E // NW,), jnp.int32),
            pltpu.VMEM((E // NW,), jnp.float32),
            pltpu.VMEM((CH,), jnp.int32),
            pltpu.VMEM((NP // NS,), jnp.float32),
            pltpu.VMEM_SHARED((NP,), jnp.float32),
        ],
    )(col, ew)


def _k2_body(x_ref, w_ref, d_ref, g1_ref, dv_ref):
    deg = d_ref[0] + d_ref[1]
    dv = jnp.where(deg > 0.0,
                   lax.rsqrt(jnp.maximum(deg, 1e-30)), 0.0)
    dv_ref[...] = dv
    h = jnp.dot(x_ref[...], w_ref[...], preferred_element_type=jnp.float32)
    g1_ref[...] = h * dv


def _k2(x_p, W1, degp):
    return pl.pallas_call(
        _k2_body,
        out_shape=[
            jax.ShapeDtypeStruct((NP, D), jnp.float32),
            jax.ShapeDtypeStruct((NP, 1), jnp.float32),
        ],
    )(x_p, W1, degp.reshape(NC, NP, 1))


def _k3_body(g1_hbm, row_hbm, col_hbm, ew_hbm, out_hbm,
             row_t, cbufs, ebufs, gbufs, acc, gsems, ssems, isems):
    cid, sid = _ids()
    ept = E // NW
    nch = ept // CH
    base = (cid * NS + sid) * ept
    pltpu.sync_copy(row_hbm.at[pl.ds(base, ept)], row_t)

    zv = jnp.zeros((16,), jnp.float32)

    def zb(e, _):
        for j in range(D // 16):
            gbufs[0][e, pl.ds(j * 16, 16)] = zv
        return 0

    lax.fori_loop(0, CH, zb, 0)
    stripe_n = NP // NS
    for k in range(stripe_n // CH):
        pltpu.sync_copy(gbufs[0], acc.at[pl.ds(sid * stripe_n + k * CH, CH)])
    plsc.subcore_barrier()

    def issue_icopy(k, b):
        off = base + k * CH
        pltpu.async_copy(col_hbm.at[pl.ds(off, CH)], cbufs[b], isems[b])
        pltpu.async_copy(ew_hbm.at[pl.ds(off, CH)],
                         ebufs[b].at[pl.ds(0, CH)], isems[b])

    def wait_icopy(b):
        pltpu.make_async_copy(col_hbm.at[pl.ds(0, CH)], cbufs[b],
                              isems[b]).wait()
        pltpu.make_async_copy(ew_hbm.at[pl.ds(0, CH)],
                              ebufs[b].at[pl.ds(0, CH)], isems[b]).wait()

    def issue_gather(k, b):
        pltpu.async_copy(g1_hbm.at[row_t.at[pl.ds(k * CH, CH)]],
                         gbufs[b], gsems[b])

    def wait_gather(b):
        pltpu.make_async_copy(g1_hbm.at[pl.ds(0, CH)], gbufs[b],
                              gsems[b]).wait()

    def issue_scatter(b):
        pltpu.async_copy(gbufs[b], acc.at[cbufs[b]], ssems[b], add=True)

    def wait_scatter(b):
        pltpu.make_async_copy(gbufs[b], acc.at[pl.ds(0, CH)], ssems[b]).wait()

    def compute(b):
        def se(eq, _):
            for u in range(4):
                e = eq * 4 + u
                ev = ebufs[b][pl.ds(e, 16)]
                cv = jnp.full((16,), ev[0], jnp.float32)
                for j in range(D // 16):
                    gbufs[b][e, pl.ds(j * 16, 16)] = (
                        gbufs[b][e, pl.ds(j * 16, 16)] * cv)
            return 0

        lax.fori_loop(0, CH // 4, se, 0)

    for b in range(NBUF - 1):
        issue_icopy(b, b)
        issue_gather(b, b)

    def slot(k, _):
        for b in range(NBUF):
            @pl.when(k % NBUF == b)
            def _():
                br = (b + NBUF - 1) % NBUF
                wait_gather(b)
                wait_icopy(b)
                compute(b)
                issue_scatter(b)

                @pl.when(k + NBUF - 1 <= nch - 1)
                def _():
                    @pl.when(k >= 1)
                    def _():
                        wait_scatter(br)
                    issue_icopy(k + NBUF - 1, br)
                    issue_gather(k + NBUF - 1, br)
        return 0

    lax.fori_loop(0, nch, slot, 0)
    for b in range(NBUF):
        wait_scatter(b)
    plsc.subcore_barrier()
    pltpu.sync_copy(acc.at[pl.ds(sid * stripe_n, stripe_n)],
                    out_hbm.at[cid, pl.ds(sid * stripe_n, stripe_n)])


def _k3(g1, row, col, ew):
    return pl.kernel(
        _k3_body,
        out_type=jax.ShapeDtypeStruct((NC, NP, D), jnp.float32),
        mesh=_mesh(),
        scratch_types=[
            pltpu.VMEM((E // NW,), jnp.int32),
            tuple(pltpu.VMEM((CH,), jnp.int32) for _ in range(NBUF)),
            tuple(pltpu.VMEM((CH + 16,), jnp.float32) for _ in range(NBUF)),
            tuple(pltpu.VMEM((CH, D), jnp.float32) for _ in range(NBUF)),
            pltpu.VMEM_SHARED((NP, D), jnp.float32),
            tuple(pltpu.SemaphoreType.DMA for _ in range(NBUF)),
            tuple(pltpu.SemaphoreType.DMA for _ in range(NBUF)),
            tuple(pltpu.SemaphoreType.DMA for _ in range(NBUF)),
        ],
    )(g1, row, col, ew)


def _k4_body(t_ref, g1_ref, dv_ref, w2_ref, b1_ref, m2_ref):
    t = t_ref[0] + t_ref[1] + g1_ref[...]
    a1 = jnp.maximum(dv_ref[...] * t + b1_ref[...], 0.0)
    h2 = jnp.sum(a1 * w2_ref[...], axis=1, keepdims=True)
    m2_ref[...] = dv_ref[...] * h2


def _k4(t1p, g1, dv, W2, b1):
    return pl.pallas_call(
        _k4_body,
        out_shape=jax.ShapeDtypeStruct((NP, 1), jnp.float32),
    )(t1p, g1, dv, W2.reshape(1, D), b1.reshape(1, D))


def _k5_body(m2_hbm, row_hbm, col_hbm, ew_hbm, out_hbm,
             row_t, col_t, ew_t, ubufs, zbuf, acc, gsems, ssems):
    cid, sid = _ids()
    ept = E // NW
    nch = ept // CH
    base = (cid * NS + sid) * ept
    pltpu.sync_copy(row_hbm.at[pl.ds(base, ept)], row_t)
    pltpu.sync_copy(col_hbm.at[pl.ds(base, ept)], col_t)
    pltpu.sync_copy(ew_hbm.at[pl.ds(base, ept)], ew_t)

    stripe_n = NP // NS
    _fill_stripe(zbuf, acc, sid * stripe_n, stripe_n, 0.0)
    plsc.subcore_barrier()

    def issue_gather(k, b):
        pltpu.async_copy(m2_hbm.at[row_t.at[pl.ds(k * CH, CH)]],
                         ubufs[b], gsems[b])

    def wait_gather(b):
        pltpu.make_async_copy(m2_hbm.at[pl.ds(0, CH)], ubufs[b],
                              gsems[b]).wait()

    def issue_scatter(k, b):
        pltpu.async_copy(ubufs[b], acc.at[col_t.at[pl.ds(k * CH, CH)]],
                         ssems[b], add=True)

    def wait_scatter(b):
        pltpu.make_async_copy(ubufs[b], acc.at[pl.ds(0, CH)],
                              ssems[b]).wait()

    def scale(k, b):
        off = k * CH
        for g in range(CH // 16):
            ubufs[b][pl.ds(g * 16, 16)] = (
                ubufs[b][pl.ds(g * 16, 16)] * ew_t[pl.ds(off + g * 16, 16)])

    for b in range(NBUF - 1):
        issue_gather(b, b)

    def slot(k, _):
        for b in range(NBUF):
            @pl.when(k % NBUF == b)
            def _():
                br = (b + NBUF - 1) % NBUF
                wait_gather(b)
                scale(k, b)
                issue_scatter(k, b)

                @pl.when(k + NBUF - 1 <= nch - 1)
                def _():
                    @pl.when(k >= 1)
                    def _():
                        wait_scatter(br)
                    issue_gather(k + NBUF - 1, br)
        return 0

    lax.fori_loop(0, nch, slot, 0)
    for b in range(NBUF):
        wait_scatter(b)
    plsc.subcore_barrier()
    pltpu.sync_copy(acc.at[pl.ds(sid * stripe_n, stripe_n)],
                    out_hbm.at[cid, pl.ds(sid * stripe_n, stripe_n)])


def _k5(m2, row, col, ew):
    return pl.kernel(
        _k5_body,
        out_type=jax.ShapeDtypeStruct((NC, NP), jnp.float32),
        mesh=_mesh(),
        scratch_types=[
            pltpu.VMEM((E // NW,), jnp.int32),
            pltpu.VMEM((E // NW,), jnp.int32),
            pltpu.VMEM((E // NW,), jnp.float32),
            tuple(pltpu.VMEM((CH,), jnp.float32) for _ in range(NBUF)),
            pltpu.VMEM((NP // NS,), jnp.float32),
            pltpu.VMEM_SHARED((NP,), jnp.float32),
            tuple(pltpu.SemaphoreType.DMA for _ in range(NBUF)),
            tuple(pltpu.SemaphoreType.DMA for _ in range(NBUF)),
        ],
    )(m2, row, col, ew)


def _k6_body(t_ref, m2_ref, dv_ref, b2_ref, o_ref):
    t = t_ref[0] + t_ref[1] + m2_ref[...]
    o_ref[...] = dv_ref[...] * t + b2_ref[0, 0]


def _k6(t2p, m2, dv, b2):
    return pl.pallas_call(
        _k6_body,
        out_shape=jax.ShapeDtypeStruct((NP, 1), jnp.float32),
    )(t2p.reshape(NC, NP, 1), m2, dv, jnp.reshape(b2, (1, 1)))


def kernel(x, edge_index, edge_weight, W1, b1, W2, b2):
    row = edge_index[0]
    col = edge_index[1]
    x_p = jnp.pad(x, ((0, NP - N), (0, 0)))

    degp = _k1(col, edge_weight)
    g1, dv = _k2(x_p, W1, degp)
    t1p = _k3(g1, row, col, edge_weight)
    m2 = _k4(t1p, g1, dv, W2, b1)
    t2p = _k5(m2.reshape(NP), row, col, edge_weight)
    outp = _k6(t2p, m2, dv, b2)
    return outp[:N].reshape(N, 1)
